# baseline (speedup 1.0000x reference)
import functools

import jax
import jax.numpy as jnp
from jax import lax
from jax.experimental import pallas as pl
from jax.experimental.pallas import tpu as pltpu
from jax.experimental.pallas import tpu_sc as plsc

BLK = 256
N_SC_WORKERS = 32


def _bdot(a, b):
    return jnp.dot(a.astype(jnp.bfloat16), b.astype(jnp.bfloat16),
                   preferred_element_type=jnp.float32)


def _gate_body(z_ref, gw_ref, gb_ref, topw_ref, idx_ref):
    z = z_ref[...]
    logits = jnp.dot(z, gw_ref[...], preferred_element_type=jnp.float32) + gb_ref[...]
    mx = jnp.max(logits, axis=1, keepdims=True)
    ex = jnp.exp(logits - mx)
    w = ex / jnp.sum(ex, axis=1, keepdims=True)

    ncol = w.shape[1]
    iota = lax.broadcasted_iota(jnp.int32, w.shape, 1)
    m1 = jnp.max(w, axis=1, keepdims=True)
    i1 = jnp.min(jnp.where(w == m1, iota, ncol), axis=1, keepdims=True)
    sel1 = iota == i1
    w_m = jnp.where(sel1, -1.0, w)
    m2 = jnp.max(w_m, axis=1, keepdims=True)
    i2 = jnp.min(jnp.where(w_m == m2, iota, ncol), axis=1, keepdims=True)
    topw_ref[...] = jnp.concatenate([m1, m2], axis=1)
    idx_ref[...] = jnp.concatenate([i1, i2], axis=1)


def _gate(z, gate_W, gate_b, blk_b):
    B, D = z.shape
    grid = (B // blk_b,)
    return pl.pallas_call(
        _gate_body,
        grid=grid,
        in_specs=[
            pl.BlockSpec((blk_b, D), lambda i: (i, 0)),
            pl.BlockSpec(gate_W.shape, lambda i: (0, 0)),
            pl.BlockSpec(gate_b.shape, lambda i: (0, 0)),
        ],
        out_specs=[
            pl.BlockSpec((blk_b, 2), lambda i: (i, 0)),
            pl.BlockSpec((blk_b, 2), lambda i: (i, 0)),
        ],
        out_shape=[
            jax.ShapeDtypeStruct((B, 2), jnp.float32),
            jax.ShapeDtypeStruct((B, 2), jnp.int32),
        ],
    )(z, gate_W, gate_b)


def _mem_body(z_ref, mk_ref, mv_ref, m_ref):
    z = z_ref[...]
    s = lax.dot_general(z.astype(jnp.bfloat16), mk_ref[...].astype(jnp.bfloat16),
                        (((1,), (1,)), ((), ())),
                        preferred_element_type=jnp.float32)
    smx = jnp.max(s, axis=1, keepdims=True)
    es = jnp.exp(s - smx)
    attn = es / jnp.sum(es, axis=1, keepdims=True)
    m_ref[...] = _bdot(attn, mv_ref[...])


def _mem(z, mem_keys, mem_values, blk_b):
    B, D = z.shape
    grid = (B // blk_b,)
    return pl.pallas_call(
        _mem_body,
        grid=grid,
        in_specs=[
            pl.BlockSpec((blk_b, D), lambda i: (i, 0)),
            pl.BlockSpec(mem_keys.shape, lambda i: (0, 0)),
            pl.BlockSpec(mem_values.shape, lambda i: (0, 0)),
        ],
        out_specs=pl.BlockSpec((blk_b, D), lambda i: (i, 0)),
        out_shape=jax.ShapeDtypeStruct((B, D), jnp.float32),
    )(z, mem_keys, mem_values)


def _cumsum_lanes(x):
    n = x.shape[1]
    s = 1
    while s < n:
        x = x + jnp.concatenate(
            [jnp.zeros((x.shape[0], s), x.dtype), x[:, :n - s]], axis=1)
        s *= 2
    return x


def _slots_body(idxT_ref, slots_ref, meta_ref, *, n_e, n_blk_meta):
    idxT = idxT_ref[...]
    B = idxT.shape[1]
    slot = jnp.zeros(idxT.shape, jnp.int32)
    base = jnp.zeros((1, 1), jnp.int32)
    starts = []
    for e in range(n_e):
        plane = (idxT == e).astype(jnp.int32)
        c = _cumsum_lanes(plane)
        row0_tot = lax.slice(c, (0, B - 1), (1, B))
        c = c + jnp.concatenate(
            [jnp.zeros((1, B), jnp.int32),
             jnp.broadcast_to(row0_tot, (1, B))], axis=0)
        count_e = lax.slice(c, (1, B - 1), (2, B))
        rank_e = c - plane
        slot = slot + plane * (rank_e + base)
        starts.append(base // BLK)
        aligned = ((count_e + BLK - 1) // BLK) * BLK
        base = base + aligned
    used = base // BLK
    iota = lax.broadcasted_iota(jnp.int32, (1, n_blk_meta), 1)
    acc = jnp.zeros((1, n_blk_meta), jnp.int32)
    for e in range(n_e):
        acc = acc + (iota >= starts[e]).astype(jnp.int32)
    eid = jnp.maximum(acc - 1, 0)
    meta_ref[...] = jnp.where(iota == 32, used, jnp.where(iota < 32, eid, 0))
    slots_ref[...] = slot


def _slots(idxT, n_e, n_blk_meta=64):
    return pl.pallas_call(
        functools.partial(_slots_body, n_e=n_e, n_blk_meta=n_blk_meta),
        grid=(1,),
        in_specs=[pl.BlockSpec(idxT.shape, lambda i: (0, 0))],
        out_specs=[
            pl.BlockSpec(idxT.shape, lambda i: (0, 0)),
            pl.BlockSpec((1, n_blk_meta), lambda i: (0, 0)),
        ],
        out_shape=[
            jax.ShapeDtypeStruct(idxT.shape, jnp.int32),
            jax.ShapeDtypeStruct((1, n_blk_meta), jnp.int32),
        ],
    )(idxT)


def _sc_dispatch(slots, z, pad_rows):
    P = slots.shape[0]
    B, D = z.shape
    ppw = P // N_SC_WORKERS
    chunk = 64
    mesh = plsc.VectorSubcoreMesh(core_axis_name="c", subcore_axis_name="s")

    @functools.partial(
        pl.kernel,
        out_type=jax.ShapeDtypeStruct((pad_rows, D), jnp.float32),
        mesh=mesh,
        scratch_types=[
            pltpu.VMEM((chunk,), jnp.int32),
            pltpu.VMEM((chunk,), jnp.int32),
            pltpu.VMEM((chunk, D), jnp.float32),
            pltpu.SemaphoreType.DMA,
        ],
    )
    def disp(slots_hbm, z_hbm, zp_hbm, tokbuf, slotbuf, rowbuf, sem):
        nc = 2
        wid = lax.axis_index("s") * nc + lax.axis_index("c")
        lane = lax.broadcasted_iota(jnp.int32, (16,), 0)
        for c in range(ppw // chunk):
            pb = wid * ppw + c * chunk
            for j in range(chunk // 16):
                pv = lane + (pb + 16 * j)
                tokbuf[pl.ds(16 * j, 16)] = jnp.where(pv >= B, pv - B, pv)
            pltpu.sync_copy(slots_hbm.at[pl.ds(pb, chunk)], slotbuf)
            pltpu.async_copy(z_hbm.at[tokbuf], rowbuf, sem).wait()
            pltpu.async_copy(rowbuf, zp_hbm.at[slotbuf], sem).wait()

    return disp(slots, z)


def _sc_combine(slots, y_pad):
    P = slots.shape[0]
    B = P // 2
    D = y_pad.shape[1]
    tpw = B // N_SC_WORKERS
    mesh = plsc.VectorSubcoreMesh(core_axis_name="c", subcore_axis_name="s")
    oshape = jax.ShapeDtypeStruct((B, D), jnp.float32)

    @functools.partial(
        pl.kernel,
        out_type=[oshape, oshape],
        mesh=mesh,
        scratch_types=[
            pltpu.VMEM((tpw,), jnp.int32),
            pltpu.VMEM((tpw, D), jnp.float32),
            pltpu.SemaphoreType.DMA,
        ],
    )
    def comb(slots_hbm, ypad_hbm, yg0_hbm, yg1_hbm, sidx, rowbuf, sem):
        nc = 2
        wid = lax.axis_index("s") * nc + lax.axis_index("c")
        tb = wid * tpw
        for k, out_hbm in ((0, yg0_hbm), (1, yg1_hbm)):
            pltpu.sync_copy(slots_hbm.at[pl.ds(k * B + tb, tpw)], sidx)
            pltpu.async_copy(ypad_hbm.at[sidx], rowbuf, sem).wait()
            pltpu.sync_copy(rowbuf, out_hbm.at[pl.ds(tb, tpw)])

    return comb(slots, y_pad)


def _experts_body(meta_ref, zp_ref, w1_ref, b1_ref, w2_ref, b2_ref, y_ref,
                  w1s_ref, w2s_ref):
    b = pl.program_id(0)
    valid = b < meta_ref[0, 32]
    prev = meta_ref[0, jnp.maximum(b - 1, 0)]
    fresh = jnp.logical_or(b == 0, meta_ref[0, b] != prev)

    @pl.when(jnp.logical_and(valid, fresh))
    def _():
        w1s_ref[...] = w1_ref[0].astype(jnp.bfloat16)
        w2s_ref[...] = w2_ref[0].astype(jnp.bfloat16)

    @pl.when(valid)
    def _():
        zb = zp_ref[...].astype(jnp.bfloat16)
        h = jnp.maximum(
            jnp.dot(zb, w1s_ref[...], preferred_element_type=jnp.float32)
            + b1_ref[0], 0.0)
        y_ref[...] = (jnp.dot(h.astype(jnp.bfloat16), w2s_ref[...],
                              preferred_element_type=jnp.float32) + b2_ref[0])


def _experts_grouped(meta, zp, e_W1, e_b1, e_W2, e_b2):
    PAD, D = zp.shape
    E, _, H = e_W1.shape
    nblk = PAD // BLK
    grid_spec = pltpu.PrefetchScalarGridSpec(
        num_scalar_prefetch=1,
        grid=(nblk,),
        in_specs=[
            pl.BlockSpec((BLK, D), lambda b, m: (b, 0)),
            pl.BlockSpec((1, D, H), lambda b, m: (m[0, b], 0, 0)),
            pl.BlockSpec((1, 1, H), lambda b, m: (m[0, b], 0, 0)),
            pl.BlockSpec((1, H, D), lambda b, m: (m[0, b], 0, 0)),
            pl.BlockSpec((1, 1, D), lambda b, m: (m[0, b], 0, 0)),
        ],
        out_specs=pl.BlockSpec((BLK, D), lambda b, m: (b, 0)),
        scratch_shapes=[
            pltpu.VMEM((D, H), jnp.bfloat16),
            pltpu.VMEM((H, D), jnp.bfloat16),
        ],
    )
    return pl.pallas_call(
        _experts_body,
        grid_spec=grid_spec,
        out_shape=jax.ShapeDtypeStruct((PAD, D), jnp.float32),
        compiler_params=pltpu.CompilerParams(
            dimension_semantics=("arbitrary",)),
    )(meta, zp, e_W1, e_b1.reshape(E, 1, H), e_W2, e_b2.reshape(E, 1, D))


def _gelu(x):
    return 0.5 * x * (1.0 + lax.erf(x * 0.7071067811865476))


def _gmlp_body(mode, *refs):
    if mode == "combine":
        m_ref, topw_ref, yg0_ref, yg1_ref = refs[:4]
        refs = refs[4:]
        w0 = topw_ref[:, 0:1]
        w1 = topw_ref[:, 1:2]
        x = m_ref[...] + w0 * yg0_ref[...] + w1 * yg1_ref[...]
    else:
        x_ref = refs[0]
        refs = refs[1:]
        x = x_ref[...]
    (fc1w_ref, fc1b_ref, fc2w_ref, fc2b_ref, gw_ref, gb_ref, out_ref,
     fc1s_ref, fc2s_ref, gs_ref) = refs

    @pl.when(pl.program_id(0) == 0)
    def _():
        fc1s_ref[...] = fc1w_ref[...].astype(jnp.bfloat16)
        fc2s_ref[...] = fc2w_ref[...].astype(jnp.bfloat16)
        gs_ref[...] = gw_ref[...].astype(jnp.bfloat16)

    xb = x.astype(jnp.bfloat16)
    h = _gelu(jnp.dot(xb, fc1s_ref[...], preferred_element_type=jnp.float32)
              + fc1b_ref[...])
    h2 = (jnp.dot(h.astype(jnp.bfloat16), fc2s_ref[...],
                  preferred_element_type=jnp.float32) + fc2b_ref[...])
    gl = jnp.dot(xb, gs_ref[...], preferred_element_type=jnp.float32) + gb_ref[...]
    g = 1.0 / (1.0 + jnp.exp(-gl))
    out_ref[...] = x + g * h2


def _gmlp(x, extra, fc1_W, fc1_b, fc2_W, fc2_b, gate_W, gate_b, blk_b):
    B, D = (extra[0].shape if extra is not None else x.shape)
    grid = (B // blk_b,)
    bspec = pl.BlockSpec((blk_b, D), lambda i: (i, 0))
    full = lambda a: pl.BlockSpec(a.shape, lambda i: (0, 0))
    mode = "combine" if extra is not None else "plain"
    ins, specs = [x], [bspec]
    if extra is not None:
        m, topw, yg0, yg1 = extra
        ins = [m, topw, yg0, yg1]
        specs = [bspec, pl.BlockSpec((blk_b, 2), lambda i: (i, 0)),
                 bspec, bspec]
    ins += [fc1_W, fc1_b, fc2_W, fc2_b, gate_W, gate_b]
    specs += [full(fc1_W), full(fc1_b), full(fc2_W), full(fc2_b),
              full(gate_W), full(gate_b)]
    H = fc1_W.shape[1]
    return pl.pallas_call(
        functools.partial(_gmlp_body, mode),
        grid=grid,
        in_specs=specs,
        out_specs=bspec,
        out_shape=jax.ShapeDtypeStruct((B, D), jnp.float32),
        scratch_shapes=[
            pltpu.VMEM((D, H), jnp.bfloat16),
            pltpu.VMEM((H, D), jnp.bfloat16),
            pltpu.VMEM((D, D), jnp.bfloat16),
        ],
        compiler_params=pltpu.CompilerParams(
            dimension_semantics=("arbitrary",)),
    )(*ins)


def _head_body(x_ref, w_ref, b_ref, out_ref):
    out_ref[...] = _bdot(x_ref[...], w_ref[...]) + b_ref[...]


def _lm_head(x, ad_W, ad_b, blk_v):
    B, D = x.shape
    V = ad_W.shape[1]
    grid = (V // blk_v,)
    return pl.pallas_call(
        _head_body,
        grid=grid,
        in_specs=[
            pl.BlockSpec((B, D), lambda v: (0, 0)),
            pl.BlockSpec((D, blk_v), lambda v: (0, v)),
            pl.BlockSpec((1, blk_v), lambda v: (0, v)),
        ],
        out_specs=pl.BlockSpec((B, blk_v), lambda v: (0, v)),
        out_shape=jax.ShapeDtypeStruct((B, V), jnp.float32),
    )(x, ad_W, ad_b)


def kernel(z, gate_W, gate_b, e_W1, e_b1, e_W2, e_b2, mem_keys, mem_values,
           g1_fc1_W, g1_fc1_b, g1_fc2_W, g1_fc2_b, g1_gate_W, g1_gate_b,
           g2_fc1_W, g2_fc1_b, g2_fc2_W, g2_fc2_b, g2_gate_W, g2_gate_b,
           ad_W, ad_b):
    B, D = z.shape
    E = gate_W.shape[1]
    blk_b = min(256, B)
    blk_v = 1280 if ad_W.shape[1] % 1280 == 0 else ad_W.shape[1]
    pad_rows = 2 * B + E * BLK

    r2 = lambda v: v.reshape(1, -1)
    topw, idx = _gate(z, gate_W, r2(gate_b), blk_b)
    slots2, meta = _slots(idx.T, E)
    slots = slots2.reshape(2 * B)
    zp = _sc_dispatch(slots, z, pad_rows)
    m = _mem(z, mem_keys, mem_values, blk_b)
    y_pad = _experts_grouped(meta, zp, e_W1, e_b1, e_W2, e_b2)
    yg0, yg1 = _sc_combine(slots, y_pad)
    x1 = _gmlp(None, (m, topw, yg0, yg1),
               g1_fc1_W, r2(g1_fc1_b), g1_fc2_W, r2(g1_fc2_b),
               g1_gate_W, r2(g1_gate_b), blk_b)
    x2 = _gmlp(x1, None, g2_fc1_W, r2(g2_fc1_b), g2_fc2_W, r2(g2_fc2_b),
               g2_gate_W, r2(g2_gate_b), blk_b)
    logits = _lm_head(x2, ad_W, r2(ad_b), blk_v)
    return (x2, logits)


# speedup vs baseline: 1.0361x; 1.0361x over previous
import functools

import jax
import jax.numpy as jnp
from jax import lax
from jax.experimental import pallas as pl
from jax.experimental.pallas import tpu as pltpu
from jax.experimental.pallas import tpu_sc as plsc

BLK = 256
N_SC_WORKERS = 32


def _router_body(z_ref, gw_ref, gb_ref, mk_ref, mv_ref,
                 topw_ref, idx_ref, m_ref):
    z = z_ref[...]
    logits = jnp.dot(z, gw_ref[...], preferred_element_type=jnp.float32) + gb_ref[...]
    mx = jnp.max(logits, axis=1, keepdims=True)
    ex = jnp.exp(logits - mx)
    w = ex / jnp.sum(ex, axis=1, keepdims=True)

    ncol = w.shape[1]
    iota = lax.broadcasted_iota(jnp.int32, w.shape, 1)
    m1 = jnp.max(w, axis=1, keepdims=True)
    i1 = jnp.min(jnp.where(w == m1, iota, ncol), axis=1, keepdims=True)
    sel1 = iota == i1
    w_m = jnp.where(sel1, -1.0, w)
    m2 = jnp.max(w_m, axis=1, keepdims=True)
    i2 = jnp.min(jnp.where(w_m == m2, iota, ncol), axis=1, keepdims=True)
    topw_ref[...] = jnp.concatenate([m1, m2], axis=1)
    idx_ref[...] = jnp.concatenate([i1, i2], axis=1)

    s = lax.dot_general(z, mk_ref[...], (((1,), (1,)), ((), ())),
                        preferred_element_type=jnp.float32)
    smx = jnp.max(s, axis=1, keepdims=True)
    es = jnp.exp(s - smx)
    attn = es / jnp.sum(es, axis=1, keepdims=True)
    m_ref[...] = jnp.dot(attn, mv_ref[...], preferred_element_type=jnp.float32)


def _router(z, gate_W, gate_b, mem_keys, mem_values, blk_b):
    B, D = z.shape
    grid = (B // blk_b,)
    return pl.pallas_call(
        _router_body,
        grid=grid,
        in_specs=[
            pl.BlockSpec((blk_b, D), lambda i: (i, 0)),
            pl.BlockSpec(gate_W.shape, lambda i: (0, 0)),
            pl.BlockSpec(gate_b.shape, lambda i: (0, 0)),
            pl.BlockSpec(mem_keys.shape, lambda i: (0, 0)),
            pl.BlockSpec(mem_values.shape, lambda i: (0, 0)),
        ],
        out_specs=[
            pl.BlockSpec((blk_b, 2), lambda i: (i, 0)),
            pl.BlockSpec((blk_b, 2), lambda i: (i, 0)),
            pl.BlockSpec((blk_b, D), lambda i: (i, 0)),
        ],
        out_shape=[
            jax.ShapeDtypeStruct((B, 2), jnp.float32),
            jax.ShapeDtypeStruct((B, 2), jnp.int32),
            jax.ShapeDtypeStruct((B, D), jnp.float32),
        ],
    )(z, gate_W, gate_b, mem_keys, mem_values)


def _cumsum_lanes(x):
    n = x.shape[1]
    s = 1
    while s < n:
        x = x + jnp.concatenate(
            [jnp.zeros((x.shape[0], s), x.dtype), x[:, :n - s]], axis=1)
        s *= 2
    return x


def _slots_body(idxT_ref, slots_ref, meta_ref, *, n_e, n_blk_meta):
    idxT = idxT_ref[...]
    B = idxT.shape[1]
    slot = jnp.zeros(idxT.shape, jnp.int32)
    base = jnp.zeros((1, 1), jnp.int32)
    starts = []
    for e in range(n_e):
        plane = (idxT == e).astype(jnp.int32)
        c = _cumsum_lanes(plane)
        row0_tot = lax.slice(c, (0, B - 1), (1, B))
        c = c + jnp.concatenate(
            [jnp.zeros((1, B), jnp.int32),
             jnp.broadcast_to(row0_tot, (1, B))], axis=0)
        count_e = lax.slice(c, (1, B - 1), (2, B))
        rank_e = c - plane
        slot = slot + plane * (rank_e + base)
        starts.append(base // BLK)
        aligned = ((count_e + BLK - 1) // BLK) * BLK
        base = base + aligned
    used = base // BLK
    iota = lax.broadcasted_iota(jnp.int32, (1, n_blk_meta), 1)
    acc = jnp.zeros((1, n_blk_meta), jnp.int32)
    for e in range(n_e):
        acc = acc + (iota >= starts[e]).astype(jnp.int32)
    eid = jnp.maximum(acc - 1, 0)
    meta_ref[...] = jnp.where(iota == 32, used, jnp.where(iota < 32, eid, 0))
    slots_ref[...] = slot


def _slots(idxT, n_e, n_blk_meta=64):
    return pl.pallas_call(
        functools.partial(_slots_body, n_e=n_e, n_blk_meta=n_blk_meta),
        grid=(1,),
        in_specs=[pl.BlockSpec(idxT.shape, lambda i: (0, 0))],
        out_specs=[
            pl.BlockSpec(idxT.shape, lambda i: (0, 0)),
            pl.BlockSpec((1, n_blk_meta), lambda i: (0, 0)),
        ],
        out_shape=[
            jax.ShapeDtypeStruct(idxT.shape, jnp.int32),
            jax.ShapeDtypeStruct((1, n_blk_meta), jnp.int32),
        ],
    )(idxT)


def _sc_dispatch(slots, z, pad_rows):
    P = slots.shape[0]
    B, D = z.shape
    ppw = P // N_SC_WORKERS
    chunk = 32
    nchunk = ppw // chunk
    mesh = plsc.VectorSubcoreMesh(core_axis_name="c", subcore_axis_name="s")

    @functools.partial(
        pl.kernel,
        out_type=jax.ShapeDtypeStruct((pad_rows, D), jnp.float32),
        mesh=mesh,
        scratch_types=[
            pltpu.VMEM((chunk,), jnp.int32),
            pltpu.VMEM((chunk,), jnp.int32),
            pltpu.VMEM((chunk,), jnp.int32),
            pltpu.VMEM((chunk,), jnp.int32),
            pltpu.VMEM((chunk, D), jnp.float32),
            pltpu.VMEM((chunk, D), jnp.float32),
            pltpu.SemaphoreType.DMA,
            pltpu.SemaphoreType.DMA,
        ],
    )
    def disp(slots_hbm, z_hbm, zp_hbm, slot0, slot1, tok0, tok1, row0, row1,
             gsem, ssem):
        nc = 2
        wid = lax.axis_index("s") * nc + lax.axis_index("c")
        lane = lax.broadcasted_iota(jnp.int32, (16,), 0)
        slotbufs = (slot0, slot1)
        toks = (tok0, tok1)
        rows = (row0, row1)
        scatters = [None] * nchunk
        for c in range(nchunk):
            sb, tb, rb = slotbufs[c % 2], toks[c % 2], rows[c % 2]
            if c >= 2:
                scatters[c - 2].wait()
            pltpu.sync_copy(slots_hbm.at[pl.ds(wid * ppw + c * chunk, chunk)],
                            sb)
            for j in range(chunk // 16):
                pv = lane + (wid * ppw + c * chunk + 16 * j)
                tb[pl.ds(16 * j, 16)] = jnp.where(pv >= B, pv - B, pv)
            pltpu.async_copy(z_hbm.at[tb], rb, gsem).wait()
            scatters[c] = pltpu.async_copy(rb, zp_hbm.at[sb], ssem)
        scatters[nchunk - 2].wait()
        scatters[nchunk - 1].wait()

    return disp(slots, z)


def _sc_combine(slots, y_pad):
    P = slots.shape[0]
    B = P // 2
    D = y_pad.shape[1]
    tpw = B // N_SC_WORKERS
    mesh = plsc.VectorSubcoreMesh(core_axis_name="c", subcore_axis_name="s")
    oshape = jax.ShapeDtypeStruct((B, D), jnp.float32)

    @functools.partial(
        pl.kernel,
        out_type=[oshape, oshape],
        mesh=mesh,
        scratch_types=[
            pltpu.VMEM((tpw // 2,), jnp.int32),
            pltpu.VMEM((tpw // 2,), jnp.int32),
            pltpu.VMEM((tpw // 2, D), jnp.float32),
            pltpu.VMEM((tpw // 2, D), jnp.float32),
            pltpu.SemaphoreType.DMA,
            pltpu.SemaphoreType.DMA,
        ],
    )
    def comb(slots_hbm, ypad_hbm, yg0_hbm, yg1_hbm, sidx0, sidx1, row0, row1,
             gsem, wsem):
        nc = 2
        ch = tpw // 2
        wid = lax.axis_index("s") * nc + lax.axis_index("c")
        tb = wid * tpw
        sidxs = (sidx0, sidx1)
        rows = (row0, row1)
        units = [(k, c) for k in (0, 1) for c in (0, 1)]
        writes = [None] * 4
        for u, (k, c) in enumerate(units):
            sb, rb = sidxs[u % 2], rows[u % 2]
            if u >= 2:
                writes[u - 2].wait()
            src = k * B + tb + c * ch
            pltpu.sync_copy(slots_hbm.at[pl.ds(src, ch)], sb)
            pltpu.async_copy(ypad_hbm.at[sb], rb, gsem).wait()
            out_hbm = yg0_hbm if k == 0 else yg1_hbm
            writes[u] = pltpu.async_copy(rb, out_hbm.at[pl.ds(tb + c * ch, ch)],
                                         wsem)
        writes[2].wait()
        writes[3].wait()

    return comb(slots, y_pad)


def _experts_body(meta_ref, zp_ref, w1_ref, b1_ref, w2_ref, b2_ref, y_ref):
    b = pl.program_id(0)

    @pl.when(b < meta_ref[0, 32])
    def _():
        h = jnp.maximum(
            jnp.dot(zp_ref[...], w1_ref[0],
                    preferred_element_type=jnp.float32) + b1_ref[0], 0.0)
        y_ref[...] = (jnp.dot(h, w2_ref[0], preferred_element_type=jnp.float32)
                      + b2_ref[0])


def _experts_grouped(meta, zp, e_W1, e_b1, e_W2, e_b2):
    PAD, D = zp.shape
    E, _, H = e_W1.shape
    nblk = PAD // BLK
    grid_spec = pltpu.PrefetchScalarGridSpec(
        num_scalar_prefetch=1,
        grid=(nblk,),
        in_specs=[
            pl.BlockSpec((BLK, D), lambda b, m: (b, 0)),
            pl.BlockSpec((1, D, H), lambda b, m: (m[0, b], 0, 0)),
            pl.BlockSpec((1, 1, H), lambda b, m: (m[0, b], 0, 0)),
            pl.BlockSpec((1, H, D), lambda b, m: (m[0, b], 0, 0)),
            pl.BlockSpec((1, 1, D), lambda b, m: (m[0, b], 0, 0)),
        ],
        out_specs=pl.BlockSpec((BLK, D), lambda b, m: (b, 0)),
    )
    return pl.pallas_call(
        _experts_body,
        grid_spec=grid_spec,
        out_shape=jax.ShapeDtypeStruct((PAD, D), jnp.float32),
        compiler_params=pltpu.CompilerParams(
            dimension_semantics=("arbitrary",)),
    )(meta, zp, e_W1, e_b1.reshape(E, 1, H), e_W2, e_b2.reshape(E, 1, D))


def _gelu(x):
    return 0.5 * x * (1.0 + lax.erf(x * 0.7071067811865476))


def _gmlp_math(x, fc1w, fc1b, fc2w, fc2b, gw, gb):
    h = _gelu(jnp.dot(x, fc1w, preferred_element_type=jnp.float32) + fc1b)
    h2 = jnp.dot(h, fc2w, preferred_element_type=jnp.float32) + fc2b
    gl = jnp.dot(x, gw, preferred_element_type=jnp.float32) + gb
    g = 1.0 / (1.0 + jnp.exp(-gl))
    return x + g * h2


def _gmlp2_body(m_ref, topw_ref, yg0_ref, yg1_ref,
                f1w_ref, f1b_ref, f2w_ref, f2b_ref, g1w_ref, g1b_ref,
                e1w_ref, e1b_ref, e2w_ref, e2b_ref, g2w_ref, g2b_ref,
                out_ref):
    w0 = topw_ref[:, 0:1]
    w1 = topw_ref[:, 1:2]
    x = m_ref[...] + w0 * yg0_ref[...] + w1 * yg1_ref[...]
    x = _gmlp_math(x, f1w_ref[...], f1b_ref[...], f2w_ref[...], f2b_ref[...],
                   g1w_ref[...], g1b_ref[...])
    out_ref[...] = _gmlp_math(x, e1w_ref[...], e1b_ref[...], e2w_ref[...],
                              e2b_ref[...], g2w_ref[...], g2b_ref[...])


def _gmlp_fused(m, topw, yg0, yg1, ws, blk_b):
    B, D = m.shape
    grid = (B // blk_b,)
    bspec = pl.BlockSpec((blk_b, D), lambda i: (i, 0))
    full = lambda a: pl.BlockSpec(a.shape, lambda i: (0, 0))
    ins = [m, topw, yg0, yg1] + list(ws)
    specs = [bspec, pl.BlockSpec((blk_b, 2), lambda i: (i, 0)), bspec, bspec]
    specs += [full(a) for a in ws]
    return pl.pallas_call(
        _gmlp2_body,
        grid=grid,
        in_specs=specs,
        out_specs=bspec,
        out_shape=jax.ShapeDtypeStruct((B, D), jnp.float32),
    )(*ins)


def _head_body(x_ref, w_ref, b_ref, out_ref):
    out_ref[...] = (jnp.dot(x_ref[...], w_ref[...],
                            preferred_element_type=jnp.float32) + b_ref[...])


def _lm_head(x, ad_W, ad_b, blk_v):
    B, D = x.shape
    V = ad_W.shape[1]
    grid = (V // blk_v,)
    return pl.pallas_call(
        _head_body,
        grid=grid,
        in_specs=[
            pl.BlockSpec((B, D), lambda v: (0, 0)),
            pl.BlockSpec((D, blk_v), lambda v: (0, v)),
            pl.BlockSpec((1, blk_v), lambda v: (0, v)),
        ],
        out_specs=pl.BlockSpec((B, blk_v), lambda v: (0, v)),
        out_shape=jax.ShapeDtypeStruct((B, V), jnp.float32),
    )(x, ad_W, ad_b)


def kernel(z, gate_W, gate_b, e_W1, e_b1, e_W2, e_b2, mem_keys, mem_values,
           g1_fc1_W, g1_fc1_b, g1_fc2_W, g1_fc2_b, g1_gate_W, g1_gate_b,
           g2_fc1_W, g2_fc1_b, g2_fc2_W, g2_fc2_b, g2_gate_W, g2_gate_b,
           ad_W, ad_b):
    B, D = z.shape
    E = gate_W.shape[1]
    blk_b = min(256, B)
    blk_v = 1280 if ad_W.shape[1] % 1280 == 0 else ad_W.shape[1]
    pad_rows = 2 * B + E * BLK

    r2 = lambda v: v.reshape(1, -1)
    topw, idx, m = _router(z, gate_W, r2(gate_b), mem_keys, mem_values, blk_b)
    slots2, meta = _slots(idx.T, E)
    slots = slots2.reshape(2 * B)
    zp = _sc_dispatch(slots, z, pad_rows)
    y_pad = _experts_grouped(meta, zp, e_W1, e_b1, e_W2, e_b2)
    yg0, yg1 = _sc_combine(slots, y_pad)
    ws = (g1_fc1_W, r2(g1_fc1_b), g1_fc2_W, r2(g1_fc2_b),
          g1_gate_W, r2(g1_gate_b),
          g2_fc1_W, r2(g2_fc1_b), g2_fc2_W, r2(g2_fc2_b),
          g2_gate_W, r2(g2_gate_b))
    x2 = _gmlp_fused(m, topw, yg0, yg1, ws, blk_b)
    logits = _lm_head(x2, ad_W, r2(ad_b), blk_v)
    return (x2, logits)


# speedup vs baseline: 1.0441x; 1.0078x over previous
import functools

import jax
import jax.numpy as jnp
from jax import lax
from jax.experimental import pallas as pl
from jax.experimental.pallas import tpu as pltpu
from jax.experimental.pallas import tpu_sc as plsc

BLK = 256
N_SC_WORKERS = 32


def _router_body(z_ref, gw_ref, gb_ref, mk_ref, mv_ref,
                 topw_ref, idx_ref, m_ref):
    z = z_ref[...]
    logits = jnp.dot(z, gw_ref[...], preferred_element_type=jnp.float32) + gb_ref[...]
    mx = jnp.max(logits, axis=1, keepdims=True)
    ex = jnp.exp(logits - mx)
    w = ex / jnp.sum(ex, axis=1, keepdims=True)

    ncol = w.shape[1]
    iota = lax.broadcasted_iota(jnp.int32, w.shape, 1)
    m1 = jnp.max(w, axis=1, keepdims=True)
    i1 = jnp.min(jnp.where(w == m1, iota, ncol), axis=1, keepdims=True)
    sel1 = iota == i1
    w_m = jnp.where(sel1, -1.0, w)
    m2 = jnp.max(w_m, axis=1, keepdims=True)
    i2 = jnp.min(jnp.where(w_m == m2, iota, ncol), axis=1, keepdims=True)
    topw_ref[...] = jnp.concatenate([m1, m2], axis=1)
    idx_ref[...] = jnp.concatenate([i1, i2], axis=1)

    s = lax.dot_general(z, mk_ref[...], (((1,), (1,)), ((), ())),
                        preferred_element_type=jnp.float32)
    smx = jnp.max(s, axis=1, keepdims=True)
    es = jnp.exp(s - smx)
    attn = es / jnp.sum(es, axis=1, keepdims=True)
    m_ref[...] = jnp.dot(attn, mv_ref[...], preferred_element_type=jnp.float32)


def _router(z, gate_W, gate_b, mem_keys, mem_values, blk_b):
    B, D = z.shape
    grid = (B // blk_b,)
    return pl.pallas_call(
        _router_body,
        grid=grid,
        in_specs=[
            pl.BlockSpec((blk_b, D), lambda i: (i, 0)),
            pl.BlockSpec(gate_W.shape, lambda i: (0, 0)),
            pl.BlockSpec(gate_b.shape, lambda i: (0, 0)),
            pl.BlockSpec(mem_keys.shape, lambda i: (0, 0)),
            pl.BlockSpec(mem_values.shape, lambda i: (0, 0)),
        ],
        out_specs=[
            pl.BlockSpec((blk_b, 2), lambda i: (i, 0)),
            pl.BlockSpec((blk_b, 2), lambda i: (i, 0)),
            pl.BlockSpec((blk_b, D), lambda i: (i, 0)),
        ],
        out_shape=[
            jax.ShapeDtypeStruct((B, 2), jnp.float32),
            jax.ShapeDtypeStruct((B, 2), jnp.int32),
            jax.ShapeDtypeStruct((B, D), jnp.float32),
        ],
    )(z, gate_W, gate_b, mem_keys, mem_values)


def _cumsum_lanes(x):
    n = x.shape[1]
    s = 1
    while s < n:
        x = x + jnp.concatenate(
            [jnp.zeros((x.shape[0], s), x.dtype), x[:, :n - s]], axis=1)
        s *= 2
    return x


def _slots_body(idxT_ref, slots_ref, meta_ref, *, n_e, n_blk_meta):
    idxT = idxT_ref[...]
    B = idxT.shape[1]
    slot = jnp.zeros(idxT.shape, jnp.int32)
    base = jnp.zeros((1, 1), jnp.int32)
    starts = []
    for e in range(n_e):
        plane = (idxT == e).astype(jnp.int32)
        c = _cumsum_lanes(plane)
        row0_tot = lax.slice(c, (0, B - 1), (1, B))
        c = c + jnp.concatenate(
            [jnp.zeros((1, B), jnp.int32),
             jnp.broadcast_to(row0_tot, (1, B))], axis=0)
        count_e = lax.slice(c, (1, B - 1), (2, B))
        rank_e = c - plane
        slot = slot + plane * (rank_e + base)
        starts.append(base // BLK)
        aligned = ((count_e + BLK - 1) // BLK) * BLK
        base = base + aligned
    used = base // BLK
    iota = lax.broadcasted_iota(jnp.int32, (1, n_blk_meta), 1)
    acc = jnp.zeros((1, n_blk_meta), jnp.int32)
    for e in range(n_e):
        acc = acc + (iota >= starts[e]).astype(jnp.int32)
    eid = jnp.maximum(acc - 1, 0)
    meta_ref[...] = jnp.where(iota == 32, used, jnp.where(iota < 32, eid, 0))
    slots_ref[...] = slot


def _slots(idxT, n_e, n_blk_meta=64):
    return pl.pallas_call(
        functools.partial(_slots_body, n_e=n_e, n_blk_meta=n_blk_meta),
        grid=(1,),
        in_specs=[pl.BlockSpec(idxT.shape, lambda i: (0, 0))],
        out_specs=[
            pl.BlockSpec(idxT.shape, lambda i: (0, 0)),
            pl.BlockSpec((1, n_blk_meta), lambda i: (0, 0)),
        ],
        out_shape=[
            jax.ShapeDtypeStruct(idxT.shape, jnp.int32),
            jax.ShapeDtypeStruct((1, n_blk_meta), jnp.int32),
        ],
    )(idxT)


def _sc_dispatch(slots, z, pad_rows):
    P = slots.shape[0]
    B, D = z.shape
    ppw = P // N_SC_WORKERS
    chunk = 32
    nchunk = ppw // chunk
    mesh = plsc.VectorSubcoreMesh(core_axis_name="c", subcore_axis_name="s")

    @functools.partial(
        pl.kernel,
        out_type=jax.ShapeDtypeStruct((pad_rows, D), jnp.float32),
        mesh=mesh,
        scratch_types=[
            pltpu.VMEM((chunk,), jnp.int32),
            pltpu.VMEM((chunk,), jnp.int32),
            pltpu.VMEM((chunk,), jnp.int32),
            pltpu.VMEM((chunk,), jnp.int32),
            pltpu.VMEM((chunk, D), jnp.float32),
            pltpu.VMEM((chunk, D), jnp.float32),
            pltpu.SemaphoreType.DMA,
            pltpu.SemaphoreType.DMA,
        ],
    )
    def disp(slots_hbm, z_hbm, zp_hbm, slot0, slot1, tok0, tok1, row0, row1,
             gsem, ssem):
        nc = 2
        wid = lax.axis_index("s") * nc + lax.axis_index("c")
        lane = lax.broadcasted_iota(jnp.int32, (16,), 0)
        slotbufs = (slot0, slot1)
        toks = (tok0, tok1)
        rows = (row0, row1)
        gathers = [None] * nchunk
        scatters = [None] * nchunk

        def start_gather(c):
            sb, tb, rb = slotbufs[c % 2], toks[c % 2], rows[c % 2]
            pltpu.sync_copy(slots_hbm.at[pl.ds(wid * ppw + c * chunk, chunk)],
                            sb)
            for j in range(chunk // 16):
                pv = lane + (wid * ppw + c * chunk + 16 * j)
                tb[pl.ds(16 * j, 16)] = jnp.where(pv >= B, pv - B, pv)
            gathers[c] = pltpu.async_copy(z_hbm.at[tb], rows[c % 2], gsem)

        start_gather(0)
        for c in range(nchunk):
            if c + 1 < nchunk:
                if c >= 1:
                    scatters[c - 1].wait()
                start_gather(c + 1)
            gathers[c].wait()
            scatters[c] = pltpu.async_copy(rows[c % 2], zp_hbm.at[slotbufs[c % 2]],
                                           ssem)
        scatters[nchunk - 2].wait()
        scatters[nchunk - 1].wait()

    return disp(slots, z)


def _sc_combine(slots, y_pad):
    P = slots.shape[0]
    B = P // 2
    D = y_pad.shape[1]
    tpw = B // N_SC_WORKERS
    mesh = plsc.VectorSubcoreMesh(core_axis_name="c", subcore_axis_name="s")
    oshape = jax.ShapeDtypeStruct((B, D), jnp.float32)

    @functools.partial(
        pl.kernel,
        out_type=[oshape, oshape],
        mesh=mesh,
        scratch_types=[
            pltpu.VMEM((tpw // 2,), jnp.int32),
            pltpu.VMEM((tpw // 2,), jnp.int32),
            pltpu.VMEM((tpw // 2, D), jnp.float32),
            pltpu.VMEM((tpw // 2, D), jnp.float32),
            pltpu.SemaphoreType.DMA,
            pltpu.SemaphoreType.DMA,
        ],
    )
    def comb(slots_hbm, ypad_hbm, yg0_hbm, yg1_hbm, sidx0, sidx1, row0, row1,
             gsem, wsem):
        nc = 2
        ch = tpw // 2
        wid = lax.axis_index("s") * nc + lax.axis_index("c")
        tb = wid * tpw
        sidxs = (sidx0, sidx1)
        rows = (row0, row1)
        units = [(k, c) for k in (0, 1) for c in (0, 1)]
        writes = [None] * 4
        for u, (k, c) in enumerate(units):
            sb, rb = sidxs[u % 2], rows[u % 2]
            if u >= 2:
                writes[u - 2].wait()
            src = k * B + tb + c * ch
            pltpu.sync_copy(slots_hbm.at[pl.ds(src, ch)], sb)
            pltpu.async_copy(ypad_hbm.at[sb], rb, gsem).wait()
            out_hbm = yg0_hbm if k == 0 else yg1_hbm
            writes[u] = pltpu.async_copy(rb, out_hbm.at[pl.ds(tb + c * ch, ch)],
                                         wsem)
        writes[2].wait()
        writes[3].wait()

    return comb(slots, y_pad)


def _experts_body(meta_ref, zp_ref, w1_ref, b1_ref, w2_ref, b2_ref, y_ref):
    b = pl.program_id(0)

    @pl.when(b < meta_ref[0, 32])
    def _():
        h = jnp.maximum(
            jnp.dot(zp_ref[...], w1_ref[0],
                    preferred_element_type=jnp.float32) + b1_ref[0], 0.0)
        y_ref[...] = (jnp.dot(h, w2_ref[0], preferred_element_type=jnp.float32)
                      + b2_ref[0])


def _experts_grouped(meta, zp, e_W1, e_b1, e_W2, e_b2):
    PAD, D = zp.shape
    E, _, H = e_W1.shape
    nblk = PAD // BLK
    grid_spec = pltpu.PrefetchScalarGridSpec(
        num_scalar_prefetch=1,
        grid=(nblk,),
        in_specs=[
            pl.BlockSpec((BLK, D), lambda b, m: (b, 0)),
            pl.BlockSpec((1, D, H), lambda b, m: (m[0, b], 0, 0)),
            pl.BlockSpec((1, 1, H), lambda b, m: (m[0, b], 0, 0)),
            pl.BlockSpec((1, H, D), lambda b, m: (m[0, b], 0, 0)),
            pl.BlockSpec((1, 1, D), lambda b, m: (m[0, b], 0, 0)),
        ],
        out_specs=pl.BlockSpec((BLK, D), lambda b, m: (b, 0)),
    )
    return pl.pallas_call(
        _experts_body,
        grid_spec=grid_spec,
        out_shape=jax.ShapeDtypeStruct((PAD, D), jnp.float32),
        compiler_params=pltpu.CompilerParams(
            dimension_semantics=("arbitrary",)),
    )(meta, zp, e_W1, e_b1.reshape(E, 1, H), e_W2, e_b2.reshape(E, 1, D))


def _gelu(x):
    return 0.5 * x * (1.0 + lax.erf(x * 0.7071067811865476))


def _gmlp_math(x, fc1w, fc1b, fc2w, fc2b, gw, gb):
    h = _gelu(jnp.dot(x, fc1w, preferred_element_type=jnp.float32) + fc1b)
    h2 = jnp.dot(h, fc2w, preferred_element_type=jnp.float32) + fc2b
    gl = jnp.dot(x, gw, preferred_element_type=jnp.float32) + gb
    g = 1.0 / (1.0 + jnp.exp(-gl))
    return x + g * h2


def _gmlp2_body(m_ref, topw_ref, yg0_ref, yg1_ref,
                f1w_ref, f1b_ref, f2w_ref, f2b_ref, g1w_ref, g1b_ref,
                e1w_ref, e1b_ref, e2w_ref, e2b_ref, g2w_ref, g2b_ref,
                out_ref):
    w0 = topw_ref[:, 0:1]
    w1 = topw_ref[:, 1:2]
    x = m_ref[...] + w0 * yg0_ref[...] + w1 * yg1_ref[...]
    x = _gmlp_math(x, f1w_ref[...], f1b_ref[...], f2w_ref[...], f2b_ref[...],
                   g1w_ref[...], g1b_ref[...])
    out_ref[...] = _gmlp_math(x, e1w_ref[...], e1b_ref[...], e2w_ref[...],
                              e2b_ref[...], g2w_ref[...], g2b_ref[...])


def _gmlp_fused(m, topw, yg0, yg1, ws, blk_b):
    B, D = m.shape
    grid = (B // blk_b,)
    bspec = pl.BlockSpec((blk_b, D), lambda i: (i, 0))
    full = lambda a: pl.BlockSpec(a.shape, lambda i: (0, 0))
    ins = [m, topw, yg0, yg1] + list(ws)
    specs = [bspec, pl.BlockSpec((blk_b, 2), lambda i: (i, 0)), bspec, bspec]
    specs += [full(a) for a in ws]
    return pl.pallas_call(
        _gmlp2_body,
        grid=grid,
        in_specs=specs,
        out_specs=bspec,
        out_shape=jax.ShapeDtypeStruct((B, D), jnp.float32),
    )(*ins)


def _head_body(x_ref, w_ref, b_ref, out_ref):
    out_ref[...] = (jnp.dot(x_ref[...], w_ref[...],
                            preferred_element_type=jnp.float32) + b_ref[...])


def _lm_head(x, ad_W, ad_b, blk_v):
    B, D = x.shape
    V = ad_W.shape[1]
    grid = (V // blk_v,)
    return pl.pallas_call(
        _head_body,
        grid=grid,
        in_specs=[
            pl.BlockSpec((B, D), lambda v: (0, 0)),
            pl.BlockSpec((D, blk_v), lambda v: (0, v)),
            pl.BlockSpec((1, blk_v), lambda v: (0, v)),
        ],
        out_specs=pl.BlockSpec((B, blk_v), lambda v: (0, v)),
        out_shape=jax.ShapeDtypeStruct((B, V), jnp.float32),
    )(x, ad_W, ad_b)


def kernel(z, gate_W, gate_b, e_W1, e_b1, e_W2, e_b2, mem_keys, mem_values,
           g1_fc1_W, g1_fc1_b, g1_fc2_W, g1_fc2_b, g1_gate_W, g1_gate_b,
           g2_fc1_W, g2_fc1_b, g2_fc2_W, g2_fc2_b, g2_gate_W, g2_gate_b,
           ad_W, ad_b):
    B, D = z.shape
    E = gate_W.shape[1]
    blk_b = min(256, B)
    blk_v = 1280 if ad_W.shape[1] % 1280 == 0 else ad_W.shape[1]
    pad_rows = 2 * B + E * BLK

    r2 = lambda v: v.reshape(1, -1)
    topw, idx, m = _router(z, gate_W, r2(gate_b), mem_keys, mem_values, blk_b)
    slots2, meta = _slots(idx.T, E)
    slots = slots2.reshape(2 * B)
    zp = _sc_dispatch(slots, z, pad_rows)
    y_pad = _experts_grouped(meta, zp, e_W1, e_b1, e_W2, e_b2)
    yg0, yg1 = _sc_combine(slots, y_pad)
    ws = (g1_fc1_W, r2(g1_fc1_b), g1_fc2_W, r2(g1_fc2_b),
          g1_gate_W, r2(g1_gate_b),
          g2_fc1_W, r2(g2_fc1_b), g2_fc2_W, r2(g2_fc2_b),
          g2_gate_W, r2(g2_gate_b))
    x2 = _gmlp_fused(m, topw, yg0, yg1, ws, blk_b)
    logits = _lm_head(x2, ad_W, r2(ad_b), blk_v)
    return (x2, logits)
